# Initial kernel scaffold; baseline (speedup 1.0000x reference)
import jax
import jax.numpy as jnp
from jax.experimental import pallas as pl


def kernel(x, edge_index, edge_types, W_et, b_et, W_ih, W_hh, b_ih, b_hh, conv1_W, conv1_b, conv2_W, conv2_b, convc1_W, convc1_b, convc2_W, convc2_b, mlp_y_W, mlp_y_b, mlp_z_W, mlp_z_b):
    raise NotImplementedError("write your pallas kernel here")


# speedup vs baseline: 11.5588x; 11.5588x over previous
import jax
import jax.numpy as jnp
from jax import lax
from jax.experimental import pallas as pl
from jax.experimental.pallas import tpu as pltpu
from jax.experimental.pallas import tpu_sc as plsc

_N = 10000
_E = 160000
_D = 128
_NT = 3
_STEPS = 6
_B = 50
_L = 200
_CC = 256

_NC = 2
_NS = 16
_NW = _NC * _NS
_CH = 128
_CHUNKS = 40
_EPT = _CH * _CHUNKS
_EPAD = _EPT * _NW
_ACC = _N + 16
_RSTRIDE = 624
_ZROWS = _ACC - 15 * _RSTRIDE
_OROWS = _N - 15 * _RSTRIDE

_BN = 1000


def _sc_body(wh_hbm, gidx_hbm, dst_hbm, zero_hbm, out_hbm,
             gidx_v, dst_v, rows_a, rows_b, acc, sem_a, sem_b):
    c = lax.axis_index("c")
    s = lax.axis_index("s")
    wid = s * _NC + c
    pltpu.sync_copy(zero_hbm, acc.at[pl.ds(s * _RSTRIDE, _ZROWS)])
    pltpu.sync_copy(gidx_hbm.at[pl.ds(wid * _CHUNKS, _CHUNKS)], gidx_v)
    pltpu.sync_copy(dst_hbm.at[pl.ds(wid * _CHUNKS, _CHUNKS)], dst_v)
    plsc.subcore_barrier()

    def chunk_pair(i, carry):
        j = 2 * i
        da = pltpu.async_copy(wh_hbm.at[gidx_v.at[j]], rows_a, sem_a)
        db = pltpu.async_copy(wh_hbm.at[gidx_v.at[j + 1]], rows_b, sem_b)
        da.wait()
        pltpu.sync_copy(rows_a, acc.at[dst_v.at[j]], add=True)
        db.wait()
        pltpu.sync_copy(rows_b, acc.at[dst_v.at[j + 1]], add=True)
        return carry

    lax.fori_loop(0, _CHUNKS // 2, chunk_pair, 0)
    plsc.subcore_barrier()
    pltpu.sync_copy(acc.at[pl.ds(s * _RSTRIDE, _OROWS)],
                    out_hbm.at[c].at[pl.ds(s * _RSTRIDE, _OROWS)])


_sc_scatter_cache = []


def _sc_scatter(wh2, gidx2, dst2, zeros_hbm):
    if not _sc_scatter_cache:
        _sc_scatter_cache.append(pl.kernel(
            _sc_body,
            out_type=jax.ShapeDtypeStruct((_NC, _N, _D), jnp.float32),
            mesh=plsc.VectorSubcoreMesh(core_axis_name="c",
                                        subcore_axis_name="s",
                                        num_cores=_NC, num_subcores=_NS),
            scratch_types=[
                pltpu.VMEM((_CHUNKS, _CH), jnp.int32),
                pltpu.VMEM((_CHUNKS, _CH), jnp.int32),
                pltpu.VMEM((_CH, _D), jnp.float32),
                pltpu.VMEM((_CH, _D), jnp.float32),
                pltpu.VMEM_SHARED((_ACC, _D), jnp.float32),
                pltpu.SemaphoreType.DMA,
                pltpu.SemaphoreType.DMA,
            ],
        ))
    return _sc_scatter_cache[0](wh2, gidx2, dst2, zeros_hbm)


def _wh_body(h_ref, wet_ref, bet_ref, wh_ref):
    h = h_ref[...]
    for t in range(_NT):
        wh_ref[t] = (jnp.dot(h, wet_ref[t], preferred_element_type=jnp.float32)
                     + bet_ref[t])


def _wh_kernel(h, wet_t, b_et):
    return pl.pallas_call(
        _wh_body,
        grid=(_N // _BN,),
        in_specs=[
            pl.BlockSpec((_BN, _D), lambda i: (i, 0)),
            pl.BlockSpec((_NT, _D, _D), lambda i: (0, 0, 0)),
            pl.BlockSpec((_NT, _D), lambda i: (0, 0)),
        ],
        out_specs=pl.BlockSpec((_NT, _BN, _D), lambda i: (0, i, 0)),
        out_shape=jax.ShapeDtypeStruct((_NT, _N, _D), jnp.float32),
    )(h, wet_t, b_et)


def _gru_body(a_ref, h_ref, wih_ref, bih_ref, whh_ref, bhh_ref, hout_ref):
    a = a_ref[0] + a_ref[1]
    h = h_ref[...]
    gi = jnp.dot(a, wih_ref[...], preferred_element_type=jnp.float32) + bih_ref[...]
    gh = jnp.dot(h, whh_ref[...], preferred_element_type=jnp.float32) + bhh_ref[...]
    r = jax.nn.sigmoid(gi[:, :_D] + gh[:, :_D])
    z = jax.nn.sigmoid(gi[:, _D:2 * _D] + gh[:, _D:2 * _D])
    n = jnp.tanh(gi[:, 2 * _D:] + r * gh[:, 2 * _D:])
    hout_ref[...] = (1.0 - z) * n + z * h


def _gru_kernel(a2, h, wih_t, bih2, whh_t, bhh2):
    return pl.pallas_call(
        _gru_body,
        grid=(_N // _BN,),
        in_specs=[
            pl.BlockSpec((_NC, _BN, _D), lambda i: (0, i, 0)),
            pl.BlockSpec((_BN, _D), lambda i: (i, 0)),
            pl.BlockSpec((_D, 3 * _D), lambda i: (0, 0)),
            pl.BlockSpec((1, 3 * _D), lambda i: (0, 0)),
            pl.BlockSpec((_D, 3 * _D), lambda i: (0, 0)),
            pl.BlockSpec((1, 3 * _D), lambda i: (0, 0)),
        ],
        out_specs=pl.BlockSpec((_BN, _D), lambda i: (i, 0)),
        out_shape=jax.ShapeDtypeStruct((_N, _D), jnp.float32),
    )(a2, h, wih_t, bih2, whh_t, bhh2)


def _branch(v, w_ref, b_ref, w2_ref, b2_ref, ch):
    t1 = jnp.dot(v[0:_L - 2], w_ref[0], preferred_element_type=jnp.float32)
    t1 = t1 + jnp.dot(v[1:_L - 1], w_ref[1], preferred_element_type=jnp.float32)
    t1 = t1 + jnp.dot(v[2:_L], w_ref[2], preferred_element_type=jnp.float32)
    t1 = jnp.maximum(t1 + b_ref[...], 0.0)
    r = t1.reshape(99, 2, ch)
    even = r[:, 0, :]
    odd = r[:, 1, :]
    m1 = jnp.maximum(jnp.maximum(even[0:98], odd[0:98]), even[1:99])
    t2 = jnp.dot(m1, w2_ref[...], preferred_element_type=jnp.float32) + b2_ref[...]
    t2 = jnp.maximum(t2, 0.0)
    r2 = t2.reshape(49, 2, ch)
    return jnp.maximum(r2[:, 0, :], r2[:, 1, :])


def _readout_body(h_ref, x_ref, w1_ref, b1_ref, w2_ref, b2_ref,
                  wc1_ref, bc1_ref, wc2_ref, bc2_ref,
                  y2_ref, z2_ref):
    hg = h_ref[0]
    xg = x_ref[0]
    cg = jnp.concatenate([hg, xg], axis=1)
    y2_ref[0] = _branch(hg, w1_ref, b1_ref, w2_ref, b2_ref, _D)
    z2_ref[0] = _branch(cg, wc1_ref, bc1_ref, wc2_ref, bc2_ref, _CC)


def _readout_kernel(h3, x3, w1t, b1, w2t, b2, wc1t, bc1, wc2t, bc2):
    return pl.pallas_call(
        _readout_body,
        grid=(_B,),
        in_specs=[
            pl.BlockSpec((1, _L, _D), lambda g: (g, 0, 0)),
            pl.BlockSpec((1, _L, _D), lambda g: (g, 0, 0)),
            pl.BlockSpec((3, _D, _D), lambda g: (0, 0, 0)),
            pl.BlockSpec((1, _D), lambda g: (0, 0)),
            pl.BlockSpec((_D, _D), lambda g: (0, 0)),
            pl.BlockSpec((1, _D), lambda g: (0, 0)),
            pl.BlockSpec((3, _CC, _CC), lambda g: (0, 0, 0)),
            pl.BlockSpec((1, _CC), lambda g: (0, 0)),
            pl.BlockSpec((_CC, _CC), lambda g: (0, 0)),
            pl.BlockSpec((1, _CC), lambda g: (0, 0)),
        ],
        out_specs=[
            pl.BlockSpec((1, 49, _D), lambda g: (g, 0, 0)),
            pl.BlockSpec((1, 49, _CC), lambda g: (g, 0, 0)),
        ],
        out_shape=[
            jax.ShapeDtypeStruct((_B, 49, _D), jnp.float32),
            jax.ShapeDtypeStruct((_B, 49, _CC), jnp.float32),
        ],
    )(h3, x3, w1t, b1, w2t, b2, wc1t, bc1, wc2t, bc2)


def _head_body(y2_ref, z2_ref, my_ref, by_ref, mz_ref, bz_ref,
               res_ref, avg_ref, temp_ref):
    y2 = y2_ref[...]
    z2 = z2_ref[...]
    yw = jnp.sum(y2 * my_ref[...][None], axis=2) + by_ref[0, 0]
    zw = jnp.sum(z2 * mz_ref[...][None], axis=2) + bz_ref[0, 0]
    avg = jnp.mean(yw * zw, axis=1, keepdims=True)
    avg_ref[...] = avg
    res_ref[...] = jax.nn.sigmoid(avg)
    temp_ref[...] = jnp.concatenate(
        [jnp.sum(y2, axis=1), jnp.sum(z2, axis=1)], axis=1)


def _head_kernel(y2, z2, my, by, mz, bz):
    return pl.pallas_call(
        _head_body,
        out_shape=[
            jax.ShapeDtypeStruct((_B, 1), jnp.float32),
            jax.ShapeDtypeStruct((_B, 1), jnp.float32),
            jax.ShapeDtypeStruct((_B, _D + _CC), jnp.float32),
        ],
    )(y2, z2, my, by, mz, bz)


def _message_pass(wh, gidx2, dst2, zeros_hbm):
    return _sc_scatter(wh.reshape(_NT * _N, _D), gidx2, dst2, zeros_hbm)


def kernel(x, edge_index, edge_types, W_et, b_et, W_ih, W_hh, b_ih, b_hh,
           conv1_W, conv1_b, conv2_W, conv2_b, convc1_W, convc1_b,
           convc2_W, convc2_b, mlp_y_W, mlp_y_b, mlp_z_W, mlp_z_b):
    src = edge_index[0]
    dst = edge_index[1]
    gidx = edge_types * _N + src
    order = jnp.argsort(dst, stable=True)
    sd = jnp.take(dst, order)
    sg = jnp.take(gidx, order)
    p = jnp.arange(1, _NW, dtype=jnp.int32) * (_E // _NW)
    prev = jnp.take(sd, p - 1)
    seg_end = jnp.searchsorted(sd, prev, side='right').astype(jnp.int32)
    b = jnp.where(jnp.take(sd, p) == prev, seg_end, p)
    b = jnp.where(b - p <= _EPT - _E // _NW, b, p)
    b_full = jnp.concatenate([jnp.zeros((1,), jnp.int32), b,
                              jnp.full((1,), _E, jnp.int32)])
    c = b_full[1:] - b_full[:-1]
    s_idx = jnp.arange(_EPT, dtype=jnp.int32)
    pos = b_full[:-1, None] + s_idx[None, :]
    valid = s_idx[None, :] < c[:, None]
    eidx = jnp.minimum(pos, _E - 1)
    flat = jnp.arange(_NW * _EPT, dtype=jnp.int32).reshape(_NW, _EPT)
    g_slots = jnp.where(valid, jnp.take(sg, eidx), (flat * 37) % (_NT * _N))
    d_slots = jnp.where(valid, jnp.take(sd, eidx), _N + (flat % 16))
    gidx2 = g_slots.reshape(_NW * _CHUNKS, _CH)
    dst2 = d_slots.reshape(_NW * _CHUNKS, _CH)
    zeros_hbm = jnp.zeros((_ZROWS, _D), jnp.float32)

    wet_t = W_et.transpose(0, 2, 1)
    wih_t = W_ih.T
    whh_t = W_hh.T
    bih2 = b_ih.reshape(1, 3 * _D)
    bhh2 = b_hh.reshape(1, 3 * _D)

    h = x
    for _unused_step in range(_STEPS):
        wh = _wh_kernel(h, wet_t, b_et)
        a2 = _message_pass(wh, gidx2, dst2, zeros_hbm)
        h = _gru_kernel(a2, h, wih_t, bih2, whh_t, bhh2)

    h3 = h.reshape(_B, _L, _D)
    x3 = x.reshape(_B, _L, _D)
    w1t = conv1_W.transpose(2, 1, 0)
    w2t = conv2_W[:, :, 0].T
    wc1t = convc1_W.transpose(2, 1, 0)
    wc2t = convc2_W[:, :, 0].T
    y2, z2 = _readout_kernel(
        h3, x3, w1t, conv1_b.reshape(1, _D), w2t, conv2_b.reshape(1, _D),
        wc1t, convc1_b.reshape(1, _CC), wc2t, convc2_b.reshape(1, _CC))
    res, avg, temp = _head_kernel(
        y2, z2, mlp_y_W, mlp_y_b.reshape(1, 1), mlp_z_W, mlp_z_b.reshape(1, 1))
    return (res.reshape(_B), avg, temp)
